# baseline (speedup 1.0000x reference)
import functools

import jax
import jax.numpy as jnp
from jax import lax
from jax.experimental import pallas as pl
from jax.experimental.pallas import tpu as pltpu
from jax.experimental.pallas import tpu_sc as plsc

F32 = jnp.float32
I32 = jnp.int32

N = 10000
NB_WORDS = 640
NPAD = 10240
NBM_ROWS = 10240
RPT = 320
ACC_ROWS = 5184
DUMMY = 5120
CAP = 16384
NZCAP = 2048
B = 128


def _p1_body(a_ref, h_ref, w1_ref, bm_ref, d_ref, xs_ref):
    a = a_ref[...]
    br = a.shape[0]
    acc = jnp.zeros((br, NB_WORDS), F32)
    for b in range(15):
        acc = acc + a[:, 640 * b:640 * (b + 1)] * float(1 << b)
    tail = jnp.concatenate(
        [a[:, 9600:10000] * float(1 << 15), jnp.zeros((br, 240), F32)], axis=1)
    acc = acc + tail
    bm_ref[...] = acc.astype(I32)
    deg = jnp.sum(a, axis=1, keepdims=True) + 1.0
    d = lax.rsqrt(deg)
    d_ref[...] = d
    xs_ref[...] = d * jnp.dot(h_ref[...], w1_ref[...],
                              preferred_element_type=F32)


def _p2_body(z1_ref, xs1_ref, d_ref, b1_ref, w2_ref, xs2_ref):
    d = d_ref[...]
    h = jax.nn.relu(d * (z1_ref[...] + xs1_ref[...]) + b1_ref[...])
    xs2_ref[...] = d * jnp.dot(h, w2_ref[...], preferred_element_type=F32)


def _p3_body(z2_ref, xs2_ref, d_ref, b2_ref, out_ref):
    o = d_ref[...] * (z2_ref[...] + xs2_ref[...]) + b2_ref[...]
    m = jnp.max(o, axis=1, keepdims=True)
    e = jnp.exp(o - m)
    out_ref[...] = e / jnp.sum(e, axis=1, keepdims=True)


def _sc_spmm_body(feat, bm_hbm, xs_hbm, z_hbm, edges_hbm, cnts_hbm,
                  strip, nzw_w, nzw_k, colflat, colstage, dststage, cbuf,
                  gbuf, obuf, acc, sem):
    c = lax.axis_index("c")
    s = lax.axis_index("s")
    tid = c * 16 + s
    rowbase = tid * RPT
    rowhi = rowbase + RPT
    accbase = s * RPT
    scbase = c * (16 * RPT)
    iota = lax.iota(I32, 16)
    zeros16 = jnp.zeros((16,), F32)

    def _zero_obuf_row(i, _):
        def _zc(k, __):
            obuf[i, pl.ds(k * 16, 16)] = zeros16
            return 0
        lax.fori_loop(0, feat // 16, _zc, 0)
        return 0
    lax.fori_loop(0, 64, _zero_obuf_row, 0)
    for k in range(5):
        pltpu.sync_copy(obuf, acc.at[pl.ds(accbase + 64 * k, 64)])

    dummy_edge = jnp.full((16,), DUMMY << 14, I32)

    def _memset(i, _):
        colflat[pl.ds(i * 16, 16)] = dummy_edge
        return 0
    lax.fori_loop(0, CAP // 16, _memset, 0)

    def _bcast(x):
        return lax.broadcast_in_dim(x, (16,), ())

    zero_v = jnp.zeros((16,), I32)
    one_v = jnp.full((16,), 1, I32)
    big_v = jnp.full((16,), 1 << 24, I32)
    ten_v = jnp.full((16,), 10, I32)
    t23_v = jnp.full((16,), 1023, I32)
    fourteen_v = jnp.full((16,), 14, I32)

    def _strip_loop(st, cnt):
        sbase = rowbase + st * 8
        pltpu.sync_copy(bm_hbm.at[pl.ds(sbase, 8)], strip)

        def _row_loop(r, nz):
            keybase = (sbase + r - scbase) * 1024

            def _chunk_loop(ch4, nz2):
                res = []
                for u in range(8):
                    ch = ch4 * 8 + u
                    wv = strip[r, pl.ds(ch * 16, 16)]
                    m = wv != zero_v
                    keyinfo = _bcast(keybase + ch * 16) + iota
                    sk = jnp.where(m, keyinfo, big_v + keyinfo)
                    ks, vs = plsc.sort_key_val(sk, wv)
                    pc = plsc.all_reduce_population_count(m)[0]
                    res.append((ks, vs, pc))
                for ks, vs, pc in res:
                    nzc = jnp.minimum(nz2, NZCAP - 16)
                    nzw_w[pl.ds(nzc, 16)] = vs
                    nzw_k[pl.ds(nzc, 16)] = ks
                    nz2 = nz2 + pc
                return nz2
            return lax.fori_loop(0, NB_WORDS // 128, _chunk_loop, nz)
        nz = lax.fori_loop(0, 8, _row_loop, jnp.int32(0))

        big29_v = jnp.full((16,), 1 << 29, I32)
        t640_v = jnp.full((16,), 640, I32)
        e23_v = jnp.full((16,), 23, I32)
        e127_v = jnp.full((16,), 127, I32)

        def _exp_loop(q, cnt2):
            wv = nzw_w[pl.ds(q * 16, 16)]
            kv = nzw_k[pl.ds(q * 16, 16)]
            lane_ok = (_bcast(q * 16) + iota) < _bcast(nz)
            dstbase = lax.shift_left(jnp.right_shift(kv, ten_v), fourteen_v)
            colbase = dstbase + jnp.bitwise_and(kv, t23_v)
            w0 = jnp.where(lane_ok, wv, zero_v)

            def _any(state):
                w, _ = state
                return plsc.all_reduce_population_count(w != zero_v)[0] > 0

            def _round(state):
                w, c2 = state
                low = jnp.bitwise_and(w, -w)
                mb = low != zero_v
                bidx = jnp.right_shift(
                    plsc.bitcast(low.astype(F32), I32), e23_v) - e127_v
                edge = colbase + bidx * t640_v
                sk = jnp.where(mb, edge, big29_v)
                vals = jnp.where(mb, edge, dummy_edge)
                _, vs = plsc.sort_key_val(sk, vals)
                pc = plsc.all_reduce_population_count(mb)[0]
                ccc = jnp.minimum(c2, CAP - 16)
                colflat[pl.ds(ccc, 16)] = vs
                return (jnp.bitwise_and(w, w - one_v), c2 + pc)

            _, cnt2 = lax.while_loop(_any, _round, (w0, cnt2))
            return cnt2
        nq = (nz + 15) // 16
        return lax.fori_loop(0, nq, _exp_loop, cnt)

    cnt = lax.fori_loop(0, 40, _strip_loop, jnp.int32(0))

    cbuf[pl.ds(0, 16)] = _bcast(cnt)
    pltpu.sync_copy(cbuf, cnts_hbm.at[tid])
    pltpu.sync_copy(colflat, edges_hbm.at[tid])

    _spmm_tail(feat, xs_hbm, z_hbm, colflat, colstage, dststage, gbuf, obuf,
               acc, sem, cnt, accbase, rowbase)


def _spmm_tail(feat, xs_hbm, z_hbm, colflat, colstage, dststage, gbuf, obuf,
               acc, sems, cnt, accbase, rowbase):
    plsc.subcore_barrier()
    nb = (jnp.minimum(cnt, CAP) + (B - 1)) // B
    t14m1_v = jnp.full((16,), (1 << 14) - 1, I32)
    fourteen_v = jnp.full((16,), 14, I32)

    def _stage(j, slot):
        def _s(k, __):
            ev = colflat[pl.ds(j * B + k * 16, 16)]
            colstage[slot, pl.ds(k * 16, 16)] = jnp.bitwise_and(ev, t14m1_v)
            dststage[slot, pl.ds(k * 16, 16)] = jnp.right_shift(ev, fourteen_v)
            return 0
        lax.fori_loop(0, B // 16, _s, 0)

    def _pair_loop(j2, _):
        _stage(2 * j2, 0)
        c0 = pltpu.async_copy(xs_hbm.at[colstage.at[0]], gbuf.at[0],
                              sems.at[0])
        _stage(2 * j2 + 1, 1)
        c1 = pltpu.async_copy(xs_hbm.at[colstage.at[1]], gbuf.at[1],
                              sems.at[1])
        c0.wait()
        pltpu.sync_copy(gbuf.at[0], acc.at[dststage.at[0]], add=True)
        c1.wait()
        pltpu.sync_copy(gbuf.at[1], acc.at[dststage.at[1]], add=True)
        return 0
    lax.fori_loop(0, (nb + 1) // 2, _pair_loop, 0)
    plsc.subcore_barrier()

    for k in range(5):
        pltpu.sync_copy(acc.at[pl.ds(accbase + 64 * k, 64)], obuf)
        pltpu.sync_copy(obuf, z_hbm.at[pl.ds(rowbase + 64 * k, 64)])


def _sc_spmm2_body(feat, xs_hbm, edges_hbm, cnts_hbm, z_hbm,
                   colflat, colstage, dststage, cbuf, gbuf, obuf, acc, sem):
    c = lax.axis_index("c")
    s = lax.axis_index("s")
    tid = c * 16 + s
    rowbase = tid * RPT
    accbase = s * RPT
    zeros16 = jnp.zeros((16,), F32)

    def _zero_obuf_row(i, _):
        def _zc(k, __):
            obuf[i, pl.ds(k * 16, 16)] = zeros16
            return 0
        lax.fori_loop(0, feat // 16, _zc, 0)
        return 0
    lax.fori_loop(0, 64, _zero_obuf_row, 0)
    for k in range(5):
        pltpu.sync_copy(obuf, acc.at[pl.ds(accbase + 64 * k, 64)])

    pltpu.sync_copy(edges_hbm.at[tid], colflat)
    pltpu.sync_copy(cnts_hbm.at[tid], cbuf)
    cnt = cbuf[pl.ds(0, 16)][0]

    _spmm_tail(feat, xs_hbm, z_hbm, colflat, colstage, dststage, gbuf, obuf,
               acc, sem, cnt, accbase, rowbase)


def _sc_mesh():
    return plsc.VectorSubcoreMesh(core_axis_name="c", subcore_axis_name="s",
                                  num_cores=2, num_subcores=16)


_SC_PARAMS = pltpu.CompilerParams(needs_layout_passes=False,
                                  use_tc_tiling_on_sc=False)


def _sc_spmm1(bm_p, xs, feat):
    kfn = functools.partial(
        pl.kernel,
        mesh=_sc_mesh(),
        compiler_params=_SC_PARAMS,
        out_type=[
            jax.ShapeDtypeStruct((NPAD, feat), F32),
            jax.ShapeDtypeStruct((32, CAP), I32),
            jax.ShapeDtypeStruct((32, 16), I32),
        ],
        scratch_types=[
            pltpu.VMEM((8, NB_WORDS), I32),
            pltpu.VMEM((NZCAP,), I32),
            pltpu.VMEM((NZCAP,), I32),
            pltpu.VMEM((CAP,), I32),
            pltpu.VMEM((2, B), I32),
            pltpu.VMEM((2, B), I32),
            pltpu.VMEM((16,), I32),
            pltpu.VMEM((2, B, feat), F32),
            pltpu.VMEM((64, feat), F32),
            pltpu.VMEM_SHARED((ACC_ROWS, feat), F32),
            pltpu.SemaphoreType.DMA((2,)),
        ],
    )(functools.partial(_sc_spmm_body, feat))
    return kfn(bm_p, xs)


def _sc_spmm2(xs, edges, cnts, feat):
    kfn = functools.partial(
        pl.kernel,
        mesh=_sc_mesh(),
        compiler_params=_SC_PARAMS,
        out_type=jax.ShapeDtypeStruct((NPAD, feat), F32),
        scratch_types=[
            pltpu.VMEM((CAP,), I32),
            pltpu.VMEM((2, B), I32),
            pltpu.VMEM((2, B), I32),
            pltpu.VMEM((16,), I32),
            pltpu.VMEM((2, B, feat), F32),
            pltpu.VMEM((64, feat), F32),
            pltpu.VMEM_SHARED((ACC_ROWS, feat), F32),
            pltpu.SemaphoreType.DMA((2,)),
        ],
    )(functools.partial(_sc_spmm2_body, feat))
    return kfn(xs, edges, cnts)


def kernel(HMatrix, adj, W1, b1, W2, b2):
    n, fin = HMatrix.shape
    hid = W1.shape[1]
    out_f = W2.shape[1]
    br = 400
    grid = (n // br,)
    params = pltpu.CompilerParams(dimension_semantics=("parallel",))
    row_blk = lambda w: pl.BlockSpec((br, w), lambda i: (i, 0))
    full_blk = lambda h, w: pl.BlockSpec((h, w), lambda i: (0, 0))

    bm, d, xs1 = pl.pallas_call(
        _p1_body,
        grid=grid,
        in_specs=[row_blk(n), row_blk(fin), full_blk(fin, hid)],
        out_specs=[row_blk(NB_WORDS), row_blk(1), row_blk(hid)],
        out_shape=[
            jax.ShapeDtypeStruct((n, NB_WORDS), I32),
            jax.ShapeDtypeStruct((n, 1), F32),
            jax.ShapeDtypeStruct((n, hid), F32),
        ],
        compiler_params=params,
    )(adj, HMatrix, W1)

    bm_p = jnp.pad(bm, ((0, NBM_ROWS - n), (0, 0)))

    z1p, edges, cnts = _sc_spmm1(bm_p, xs1, hid)
    z1 = z1p[:n]

    xs2 = pl.pallas_call(
        _p2_body,
        grid=grid,
        in_specs=[row_blk(hid), row_blk(hid), row_blk(1),
                  full_blk(1, hid), full_blk(hid, out_f)],
        out_specs=row_blk(out_f),
        out_shape=jax.ShapeDtypeStruct((n, out_f), F32),
        compiler_params=params,
    )(z1, xs1, d, b1.reshape(1, hid), W2)

    z2 = _sc_spmm2(xs2, edges, cnts, out_f)[:n]

    out = pl.pallas_call(
        _p3_body,
        grid=grid,
        in_specs=[row_blk(out_f), row_blk(out_f), row_blk(1),
                  full_blk(1, out_f)],
        out_specs=row_blk(out_f),
        out_shape=jax.ShapeDtypeStruct((n, out_f), F32),
        compiler_params=params,
    )(z2, xs2, d, b2.reshape(1, out_f))

    return out


# speedup vs baseline: 1.1252x; 1.1252x over previous
import functools

import jax
import jax.numpy as jnp
from jax import lax
from jax.experimental import pallas as pl
from jax.experimental.pallas import tpu as pltpu
from jax.experimental.pallas import tpu_sc as plsc

F32 = jnp.float32
I32 = jnp.int32

N = 10000
NB_WORDS = 640
NPAD = 10240
NBM_ROWS = 10240
RPT = 320
ACC_ROWS = 5184
DUMMY = 5120
CAP = 16384
NZCAP = 2048
B = 128


def _p1_body(a_ref, h_ref, w1_ref, bm_ref, d_ref, xs_ref):
    a = a_ref[...]
    br = a.shape[0]
    acc = jnp.zeros((br, NB_WORDS), F32)
    for b in range(15):
        acc = acc + a[:, 640 * b:640 * (b + 1)] * float(1 << b)
    tail = jnp.concatenate(
        [a[:, 9600:10000] * float(1 << 15), jnp.zeros((br, 240), F32)], axis=1)
    acc = acc + tail
    bm_ref[...] = acc.astype(I32)
    deg = jnp.sum(a, axis=1, keepdims=True) + 1.0
    d = lax.rsqrt(deg)
    d_ref[...] = d
    xs_ref[...] = d * jnp.dot(h_ref[...], w1_ref[...],
                              preferred_element_type=F32)


def _p2_body(z1_ref, xs1_ref, d_ref, b1_ref, w2_ref, xs2_ref):
    d = d_ref[...]
    h = jax.nn.relu(d * (z1_ref[...] + xs1_ref[...]) + b1_ref[...])
    xs2_ref[...] = d * jnp.dot(h, w2_ref[...], preferred_element_type=F32)


def _p3_body(z2_ref, xs2_ref, d_ref, b2_ref, out_ref):
    o = d_ref[...] * (z2_ref[...] + xs2_ref[...]) + b2_ref[...]
    m = jnp.max(o, axis=1, keepdims=True)
    e = jnp.exp(o - m)
    out_ref[...] = e / jnp.sum(e, axis=1, keepdims=True)


def _sc_spmm_body(feat, bm_hbm, xs_hbm, z_hbm, edges_hbm, cnts_hbm,
                  strip, nzw_w, nzw_k, colflat, colstage, dststage, cbuf,
                  gbuf, obuf, acc, sem):
    c = lax.axis_index("c")
    s = lax.axis_index("s")
    tid = c * 16 + s
    rowbase = tid * RPT
    rowhi = rowbase + RPT
    accbase = s * RPT
    scbase = c * (16 * RPT)
    iota = lax.iota(I32, 16)
    zeros16 = jnp.zeros((16,), F32)

    def _zero_obuf_row(i, _):
        def _zc(k, __):
            obuf[i, pl.ds(k * 16, 16)] = zeros16
            return 0
        lax.fori_loop(0, feat // 16, _zc, 0)
        return 0
    lax.fori_loop(0, 64, _zero_obuf_row, 0)
    for k in range(5):
        pltpu.sync_copy(obuf, acc.at[pl.ds(accbase + 64 * k, 64)])

    dummy_edge = jnp.full((16,), DUMMY << 14, I32)

    def _memset(i, _):
        colflat[pl.ds(i * 16, 16)] = dummy_edge
        return 0
    lax.fori_loop(0, CAP // 16, _memset, 0)

    def _bcast(x):
        return lax.broadcast_in_dim(x, (16,), ())

    zero_v = jnp.zeros((16,), I32)
    one_v = jnp.full((16,), 1, I32)
    big_v = jnp.full((16,), 1 << 24, I32)
    ten_v = jnp.full((16,), 10, I32)
    t23_v = jnp.full((16,), 1023, I32)
    fourteen_v = jnp.full((16,), 14, I32)

    def _strip_loop(st, cnt):
        sbase = rowbase + st * 8
        pltpu.sync_copy(bm_hbm.at[pl.ds(sbase, 8)], strip)

        def _row_loop(r, nz):
            keybase = (sbase + r - scbase) * 1024

            def _chunk_loop(ch4, nz2):
                res = []
                for u in range(8):
                    ch = ch4 * 8 + u
                    wv = strip[r, pl.ds(ch * 16, 16)]
                    m = wv != zero_v
                    keyinfo = _bcast(keybase + ch * 16) + iota
                    sk = jnp.where(m, keyinfo, big_v + keyinfo)
                    ks, vs = plsc.sort_key_val(sk, wv)
                    pc = plsc.all_reduce_population_count(m)[0]
                    res.append((ks, vs, pc))
                for ks, vs, pc in res:
                    nzc = jnp.minimum(nz2, NZCAP - 16)
                    nzw_w[pl.ds(nzc, 16)] = vs
                    nzw_k[pl.ds(nzc, 16)] = ks
                    nz2 = nz2 + pc
                return nz2
            return lax.fori_loop(0, NB_WORDS // 128, _chunk_loop, nz)
        nz = lax.fori_loop(0, 8, _row_loop, jnp.int32(0))

        big29_v = jnp.full((16,), 1 << 29, I32)
        t640_v = jnp.full((16,), 640, I32)
        e23_v = jnp.full((16,), 23, I32)
        e127_v = jnp.full((16,), 127, I32)

        def _exp_loop(q, cnt2):
            wv = nzw_w[pl.ds(q * 16, 16)]
            kv = nzw_k[pl.ds(q * 16, 16)]
            lane_ok = (_bcast(q * 16) + iota) < _bcast(nz)
            dstbase = lax.shift_left(jnp.right_shift(kv, ten_v), fourteen_v)
            colbase = dstbase + jnp.bitwise_and(kv, t23_v)
            w0 = jnp.where(lane_ok, wv, zero_v)

            def _any(state):
                w, _ = state
                return plsc.all_reduce_population_count(w != zero_v)[0] > 0

            def _round(state):
                w, c2 = state
                low = jnp.bitwise_and(w, -w)
                mb = low != zero_v
                bidx = jnp.right_shift(
                    plsc.bitcast(low.astype(F32), I32), e23_v) - e127_v
                edge = colbase + bidx * t640_v
                sk = jnp.where(mb, edge, big29_v)
                vals = jnp.where(mb, edge, dummy_edge)
                _, vs = plsc.sort_key_val(sk, vals)
                pc = plsc.all_reduce_population_count(mb)[0]
                ccc = jnp.minimum(c2, CAP - 16)
                colflat[pl.ds(ccc, 16)] = vs
                return (jnp.bitwise_and(w, w - one_v), c2 + pc)

            _, cnt2 = lax.while_loop(_any, _round, (w0, cnt2))
            return cnt2
        nq = (nz + 15) // 16
        return lax.fori_loop(0, nq, _exp_loop, cnt)

    cnt = lax.fori_loop(0, 40, _strip_loop, jnp.int32(0))

    cbuf[pl.ds(0, 16)] = _bcast(cnt)
    pltpu.sync_copy(cbuf, cnts_hbm.at[tid])
    pltpu.sync_copy(colflat, edges_hbm.at[tid])

    _spmm_tail(feat, xs_hbm, z_hbm, colflat, colstage, dststage, gbuf, obuf,
               acc, sem, cnt, accbase, rowbase)


def _spmm_tail(feat, xs_hbm, z_hbm, colflat, colstage, dststage, gbuf, obuf,
               acc, sems, cnt, accbase, rowbase):
    plsc.subcore_barrier()
    nb = (jnp.minimum(cnt, CAP) + (B - 1)) // B
    t14m1_v = jnp.full((16,), (1 << 14) - 1, I32)
    fourteen_v = jnp.full((16,), 14, I32)

    def _batch_loop(j, _):
        def _s(k, __):
            ev = colflat[pl.ds(j * B + k * 16, 16)]
            colstage[0, pl.ds(k * 16, 16)] = jnp.bitwise_and(ev, t14m1_v)
            dststage[0, pl.ds(k * 16, 16)] = jnp.right_shift(ev, fourteen_v)
            return 0
        lax.fori_loop(0, B // 16, _s, 0)
        pltpu.async_copy(xs_hbm.at[colstage.at[0]], gbuf.at[0],
                         sems.at[0]).wait()
        pltpu.sync_copy(gbuf.at[0], acc.at[dststage.at[0]], add=True)
        return 0
    lax.fori_loop(0, nb, _batch_loop, 0)
    plsc.subcore_barrier()

    for k in range(5):
        pltpu.sync_copy(acc.at[pl.ds(accbase + 64 * k, 64)], obuf)
        pltpu.sync_copy(obuf, z_hbm.at[pl.ds(rowbase + 64 * k, 64)])


def _sc_spmm2_body(feat, xs_hbm, edges_hbm, cnts_hbm, z_hbm,
                   colflat, colstage, dststage, cbuf, gbuf, obuf, acc, sem):
    c = lax.axis_index("c")
    s = lax.axis_index("s")
    tid = c * 16 + s
    rowbase = tid * RPT
    accbase = s * RPT
    zeros16 = jnp.zeros((16,), F32)

    def _zero_obuf_row(i, _):
        def _zc(k, __):
            obuf[i, pl.ds(k * 16, 16)] = zeros16
            return 0
        lax.fori_loop(0, feat // 16, _zc, 0)
        return 0
    lax.fori_loop(0, 64, _zero_obuf_row, 0)
    for k in range(5):
        pltpu.sync_copy(obuf, acc.at[pl.ds(accbase + 64 * k, 64)])

    pltpu.sync_copy(edges_hbm.at[tid], colflat)
    pltpu.sync_copy(cnts_hbm.at[tid], cbuf)
    cnt = cbuf[pl.ds(0, 16)][0]

    _spmm_tail(feat, xs_hbm, z_hbm, colflat, colstage, dststage, gbuf, obuf,
               acc, sem, cnt, accbase, rowbase)


def _sc_mesh():
    return plsc.VectorSubcoreMesh(core_axis_name="c", subcore_axis_name="s",
                                  num_cores=2, num_subcores=16)


_SC_PARAMS = pltpu.CompilerParams(needs_layout_passes=False,
                                  use_tc_tiling_on_sc=False)


def _sc_spmm1(bm_p, xs, feat):
    kfn = functools.partial(
        pl.kernel,
        mesh=_sc_mesh(),
        compiler_params=_SC_PARAMS,
        out_type=[
            jax.ShapeDtypeStruct((NPAD, feat), F32),
            jax.ShapeDtypeStruct((32, CAP), I32),
            jax.ShapeDtypeStruct((32, 16), I32),
        ],
        scratch_types=[
            pltpu.VMEM((8, NB_WORDS), I32),
            pltpu.VMEM((NZCAP,), I32),
            pltpu.VMEM((NZCAP,), I32),
            pltpu.VMEM((CAP,), I32),
            pltpu.VMEM((2, B), I32),
            pltpu.VMEM((2, B), I32),
            pltpu.VMEM((16,), I32),
            pltpu.VMEM((2, B, feat), F32),
            pltpu.VMEM((64, feat), F32),
            pltpu.VMEM_SHARED((ACC_ROWS, feat), F32),
            pltpu.SemaphoreType.DMA((2,)),
        ],
    )(functools.partial(_sc_spmm_body, feat))
    return kfn(bm_p, xs)


def _sc_spmm2(xs, edges, cnts, feat):
    kfn = functools.partial(
        pl.kernel,
        mesh=_sc_mesh(),
        compiler_params=_SC_PARAMS,
        out_type=jax.ShapeDtypeStruct((NPAD, feat), F32),
        scratch_types=[
            pltpu.VMEM((CAP,), I32),
            pltpu.VMEM((2, B), I32),
            pltpu.VMEM((2, B), I32),
            pltpu.VMEM((16,), I32),
            pltpu.VMEM((2, B, feat), F32),
            pltpu.VMEM((64, feat), F32),
            pltpu.VMEM_SHARED((ACC_ROWS, feat), F32),
            pltpu.SemaphoreType.DMA((2,)),
        ],
    )(functools.partial(_sc_spmm2_body, feat))
    return kfn(xs, edges, cnts)


def kernel(HMatrix, adj, W1, b1, W2, b2):
    n, fin = HMatrix.shape
    hid = W1.shape[1]
    out_f = W2.shape[1]
    br = 400
    grid = (n // br,)
    params = pltpu.CompilerParams(dimension_semantics=("parallel",))
    row_blk = lambda w: pl.BlockSpec((br, w), lambda i: (i, 0))
    full_blk = lambda h, w: pl.BlockSpec((h, w), lambda i: (0, 0))

    bm, d, xs1 = pl.pallas_call(
        _p1_body,
        grid=grid,
        in_specs=[row_blk(n), row_blk(fin), full_blk(fin, hid)],
        out_specs=[row_blk(NB_WORDS), row_blk(1), row_blk(hid)],
        out_shape=[
            jax.ShapeDtypeStruct((n, NB_WORDS), I32),
            jax.ShapeDtypeStruct((n, 1), F32),
            jax.ShapeDtypeStruct((n, hid), F32),
        ],
        compiler_params=params,
    )(adj, HMatrix, W1)

    bm_p = jnp.pad(bm, ((0, NBM_ROWS - n), (0, 0)))

    z1p, edges, cnts = _sc_spmm1(bm_p, xs1, hid)
    z1 = z1p[:n]

    xs2 = pl.pallas_call(
        _p2_body,
        grid=grid,
        in_specs=[row_blk(hid), row_blk(hid), row_blk(1),
                  full_blk(1, hid), full_blk(hid, out_f)],
        out_specs=row_blk(out_f),
        out_shape=jax.ShapeDtypeStruct((n, out_f), F32),
        compiler_params=params,
    )(z1, xs1, d, b1.reshape(1, hid), W2)

    z2 = _sc_spmm2(xs2, edges, cnts, out_f)[:n]

    out = pl.pallas_call(
        _p3_body,
        grid=grid,
        in_specs=[row_blk(out_f), row_blk(out_f), row_blk(1),
                  full_blk(1, out_f)],
        out_specs=row_blk(out_f),
        out_shape=jax.ShapeDtypeStruct((n, out_f), F32),
        compiler_params=params,
    )(z2, xs2, d, b2.reshape(1, out_f))

    return out
